# baseline (speedup 1.0000x reference)
import functools

import jax
import jax.numpy as jnp
from jax import lax
from jax.experimental import pallas as pl
from jax.experimental.pallas import tpu as pltpu
from jax.experimental.pallas import tpu_sc as plsc

_W = 0.3
_BLOCK_ROWS = 4000

_M, _D = 100000, 128
_NW = 32
_ROWS_PER_W = _M // _NW
_CHUNK_ROWS = 125
_CE = _CHUNK_ROWS * _D
_NCHUNK = _ROWS_PER_W // _CHUNK_ROWS
_NVEC = _CE // 16


def _tc_body(c_ref, h_ref, o_ref):
    o_ref[...] = (1.0 - _W) * c_ref[...] + _W * h_ref[...]


def _tc_fuse(core, hg):
    M, D = core.shape
    spec = pl.BlockSpec((_BLOCK_ROWS, D), lambda i: (i, 0))
    return pl.pallas_call(
        _tc_body,
        grid=(M // _BLOCK_ROWS,),
        in_specs=[spec, spec],
        out_specs=spec,
        out_shape=jax.ShapeDtypeStruct((M, D), core.dtype),
    )(core, hg)


_sc_mesh = plsc.VectorSubcoreMesh(core_axis_name="c", subcore_axis_name="s")


@functools.partial(
    pl.kernel,
    out_type=jax.ShapeDtypeStruct((_M * _D,), jnp.float32),
    mesh=_sc_mesh,
    scratch_types=[
        pltpu.VMEM((_CE,), jnp.float32),
        pltpu.VMEM((_CE,), jnp.float32),
    ],
)
def _sc_fuse(core_hbm, hg_hbm, out_hbm, a_buf, b_buf):
    wid = lax.axis_index("s") * 2 + lax.axis_index("c")
    wbase = wid * (_ROWS_PER_W * _D)

    def chunk_body(t, carry):
        base = wbase + t * _CE
        pltpu.sync_copy(core_hbm.at[pl.ds(base, _CE)], a_buf)
        pltpu.sync_copy(hg_hbm.at[pl.ds(base, _CE)], b_buf)

        def inner(i, c):
            sl = pl.ds(i * 16, 16)
            a_buf[sl] = (1.0 - _W) * a_buf[sl] + _W * b_buf[sl]
            return c

        lax.fori_loop(0, _NVEC, inner, 0)
        pltpu.sync_copy(a_buf, out_hbm.at[pl.ds(base, _CE)])
        return carry

    lax.fori_loop(0, _NCHUNK, chunk_body, 0)


def kernel(core_user_emb, core_item_emb, hg_user_emb, hg_item_emb):
    out_user = _tc_fuse(core_user_emb, hg_user_emb)
    out_item = _sc_fuse(
        core_item_emb.reshape(-1), hg_item_emb.reshape(-1)
    ).reshape(_M, _D)
    return (out_user, out_item)


# speedup vs baseline: 1.7943x; 1.7943x over previous
import functools

import jax
import jax.numpy as jnp
from jax import lax
from jax.experimental import pallas as pl
from jax.experimental.pallas import tpu as pltpu
from jax.experimental.pallas import tpu_sc as plsc

_W = 0.3
_M, _D = 100000, 128

_BLOCK_A = 4000
_BLOCK_B = 2000
_K = 50000

_NW = 32
_EPW = (_M - _K) * _D // _NW
_CE = 20000
_NCHUNK = _EPW // _CE
_NBUF = 2
_UNROLL = 10
_NVEC = _CE // 16


def _tc_body(c_ref, h_ref, o_ref):
    o_ref[...] = (1.0 - _W) * c_ref[...] + _W * h_ref[...]


def _tc_fuse(core, hg):
    spec = pl.BlockSpec((_BLOCK_A, _D), lambda i: (i, 0))
    return pl.pallas_call(
        _tc_body,
        grid=(_M // _BLOCK_A,),
        in_specs=[spec, spec],
        out_specs=spec,
        out_shape=jax.ShapeDtypeStruct((_M, _D), core.dtype),
    )(core, hg)


def _tc_patch_body(c_ref, h_ref, _sc_ref, o_ref):
    o_ref[...] = (1.0 - _W) * c_ref[...] + _W * h_ref[...]


def _tc_patch(core, hg, sc_out):
    spec = pl.BlockSpec((_BLOCK_B, _D), lambda i: (i, 0))
    return pl.pallas_call(
        _tc_patch_body,
        grid=(_K // _BLOCK_B,),
        in_specs=[spec, spec, pl.BlockSpec(memory_space=pl.ANY)],
        out_specs=spec,
        out_shape=jax.ShapeDtypeStruct((_M, _D), core.dtype),
        input_output_aliases={2: 0},
    )(core, hg, sc_out)


_sc_mesh = plsc.VectorSubcoreMesh(core_axis_name="c", subcore_axis_name="s")


@functools.partial(
    pl.kernel,
    out_type=jax.ShapeDtypeStruct((_M * _D,), jnp.float32),
    mesh=_sc_mesh,
    scratch_types=(
        [pltpu.VMEM((_CE,), jnp.float32) for _ in range(3 * _NBUF)]
        + [pltpu.SemaphoreType.DMA, pltpu.SemaphoreType.DMA,
           pltpu.SemaphoreType.DMA, pltpu.SemaphoreType.DMA]
    ),
)
def _sc_fuse(core_hbm, hg_hbm, out_hbm, a0, b0, o0, a1, b1, o1,
             in_sem0, in_sem1, out_sem0, out_sem1):
    a_bufs, b_bufs, o_bufs = (a0, a1), (b0, b1), (o0, o1)
    in_sems, out_sems = (in_sem0, in_sem1), (out_sem0, out_sem1)
    wid = lax.axis_index("s") * 2 + lax.axis_index("c")
    base = _K * _D + wid * _EPW

    def start_in(g):
        off = base + g * _CE
        k = g % _NBUF
        ha = pltpu.async_copy(core_hbm.at[pl.ds(off, _CE)], a_bufs[k], in_sems[k])
        hb = pltpu.async_copy(hg_hbm.at[pl.ds(off, _CE)], b_bufs[k], in_sems[k])
        return ha, hb

    handles_in = {}
    handles_out = {}
    for k in range(_NBUF):
        handles_in[k] = start_in(k)

    for g in range(_NCHUNK):
        k = g % _NBUF
        ha, hb = handles_in[k]
        ha.wait()
        hb.wait()
        if g >= _NBUF:
            handles_out[k].wait()
        a_ref, b_ref, o_ref = a_bufs[k], b_bufs[k], o_bufs[k]

        def inner(i, c):
            for u in range(_UNROLL):
                sl = pl.ds(i * (16 * _UNROLL) + u * 16, 16)
                o_ref[sl] = (1.0 - _W) * a_ref[sl] + _W * b_ref[sl]
            return c

        lax.fori_loop(0, _NVEC // _UNROLL, inner, 0, unroll=False)

        off = base + g * _CE
        handles_out[k] = pltpu.async_copy(
            o_bufs[k], out_hbm.at[pl.ds(off, _CE)], out_sems[k]
        )
        if g + _NBUF < _NCHUNK:
            handles_in[k] = start_in(g + _NBUF)

    for k in range(min(_NBUF, _NCHUNK)):
        handles_out[k].wait()


def kernel(core_user_emb, core_item_emb, hg_user_emb, hg_item_emb):
    sc_item = _sc_fuse(
        core_item_emb.reshape(-1), hg_item_emb.reshape(-1)
    ).reshape(_M, _D)
    out_user = _tc_fuse(core_user_emb, hg_user_emb)
    out_item = _tc_patch(core_item_emb, hg_item_emb, sc_item)
    return (out_user, out_item)


# speedup vs baseline: 2.7119x; 1.5114x over previous
import jax
import jax.numpy as jnp
from jax.experimental import pallas as pl

_W = 0.3
_BLOCK_ROWS = 10000


def _fuse_kernel(cu_ref, ci_ref, hu_ref, hi_ref, ou_ref, oi_ref):
    ou_ref[...] = (1.0 - _W) * cu_ref[...] + _W * hu_ref[...]
    oi_ref[...] = (1.0 - _W) * ci_ref[...] + _W * hi_ref[...]


def kernel(core_user_emb, core_item_emb, hg_user_emb, hg_item_emb):
    M, D = core_user_emb.shape
    grid = (M // _BLOCK_ROWS,)
    spec = pl.BlockSpec((_BLOCK_ROWS, D), lambda i: (i, 0))
    out_user, out_item = pl.pallas_call(
        _fuse_kernel,
        grid=grid,
        in_specs=[spec, spec, spec, spec],
        out_specs=[spec, spec],
        out_shape=[
            jax.ShapeDtypeStruct((M, D), core_user_emb.dtype),
            jax.ShapeDtypeStruct((M, D), core_item_emb.dtype),
        ],
    )(core_user_emb, core_item_emb, hg_user_emb, hg_item_emb)
    return (out_user, out_item)
